# baseline (speedup 1.0000x reference)
import functools

import jax
import jax.numpy as jnp
from jax import lax
from jax.experimental import pallas as pl
from jax.experimental.pallas import tpu as pltpu
from jax.experimental.pallas import tpu_sc as plsc

B = 16384
K = 64
NC = 2
NS = 16
NW = NC * NS
RPW = B // NW
CH = 128
NCHUNK = RPW // CH

_mesh = plsc.VectorSubcoreMesh(core_axis_name="c", subcore_axis_name="s")


@functools.partial(
    pl.kernel,
    mesh=_mesh,
    out_type=jax.ShapeDtypeStruct((B,), jnp.float32),
    scratch_types=[
        pltpu.VMEM((2, CH * K), jnp.float32),
        pltpu.VMEM((2, CH * K), jnp.float32),
        pltpu.VMEM((2, CH * K), jnp.float32),
        pltpu.VMEM((2, CH * K), jnp.float32),
        pltpu.VMEM((RPW,), jnp.float32),
        pltpu.VMEM((RPW,), jnp.float32),
        pltpu.VMEM((RPW,), jnp.float32),
        pltpu.VMEM((RPW,), jnp.float32),
        pltpu.VMEM((RPW,), jnp.float32),
        pltpu.VMEM((16,), jnp.float32),
        pltpu.SemaphoreType.DMA,
        pltpu.SemaphoreType.DMA,
    ],
)
def _sc_kernel(gu_h, gi_h, gut_h, git_h, bu_h, bi_h, but_h, bit_h, mu_h,
               out_h, gu_v, gi_v, gut_v, git_v,
               bu_v, bi_v, but_v, bit_v, out_v, mu_v, sem0, sem1):
    wid = lax.axis_index("s") * NC + lax.axis_index("c")
    row0 = wid * RPW
    sems = (sem0, sem1)

    pltpu.sync_copy(bu_h.at[pl.ds(row0, RPW)], bu_v)
    pltpu.sync_copy(bi_h.at[pl.ds(row0, RPW)], bi_v)
    pltpu.sync_copy(but_h.at[pl.ds(row0, RPW)], but_v)
    pltpu.sync_copy(bit_h.at[pl.ds(row0, RPW)], bit_v)
    pltpu.sync_copy(mu_h, mu_v)

    iota16 = lax.iota(jnp.int32, 16)
    perms = [jnp.bitwise_xor(iota16, d) for d in (8, 4, 2, 1)]
    onehots = [jnp.where(iota16 == j, 1.0, 0.0) for j in range(16)]

    def start_chunk(c):
        s = c % 2
        off = (row0 + c * CH) * K
        sem = sems[s]
        return [
            pltpu.async_copy(gu_h.at[pl.ds(off, CH * K)], gu_v.at[s], sem),
            pltpu.async_copy(gi_h.at[pl.ds(off, CH * K)], gi_v.at[s], sem),
            pltpu.async_copy(gut_h.at[pl.ds(off, CH * K)], gut_v.at[s], sem),
            pltpu.async_copy(git_h.at[pl.ds(off, CH * K)], git_v.at[s], sem),
        ]

    pending = {0: start_chunk(0)}

    for c in range(NCHUNK):
        s = c % 2
        if c + 1 < NCHUNK:
            pending[c + 1] = start_chunk(c + 1)
        for h in pending.pop(c):
            h.wait()

        @plsc.parallel_loop(0, CH // 16, 1, unroll=2)
        def grp_body(g):
            masked = []
            for j in range(16):
                acc = None
                base = (g * 16 + j) * K
                for m in range(K // 16):
                    sl = pl.ds(base + m * 16, 16)
                    p = (gu_v[s, sl] * gi_v[s, sl]
                         + gut_v[s, sl] * git_v[s, sl])
                    acc = p if acc is None else acc + p
                for pm in perms:
                    acc = acc + acc.at[pm].get(mode="promise_in_bounds")
                masked.append(acc * onehots[j])
            while len(masked) > 1:
                masked = [a + b for a, b in zip(masked[::2], masked[1::2])]
            out_v[pl.ds(c * CH + g * 16, 16)] = masked[0]

    mu = mu_v[pl.ds(0, 16)]
    for q in range(RPW // 16):
        sl = pl.ds(q * 16, 16)
        out_v[sl] = (out_v[sl] + bu_v[sl] + bi_v[sl] + but_v[sl]
                     + bit_v[sl] + mu)

    pltpu.sync_copy(out_v, out_h.at[pl.ds(row0, RPW)])


def kernel(gu, gi, gut, git, bu, bi, but, bit, Mu):
    mu1 = jnp.broadcast_to(Mu.reshape(1), (16,))
    return _sc_kernel(gu.reshape(B * K), gi.reshape(B * K),
                      gut.reshape(B * K), git.reshape(B * K),
                      bu.reshape(B), bi.reshape(B),
                      but.reshape(B), bit.reshape(B), mu1)


# speedup vs baseline: 1.3223x; 1.3223x over previous
import jax
import jax.numpy as jnp
from jax.experimental import pallas as pl
from jax.experimental.pallas import tpu as pltpu

B = 16384
K = 64
GRID = 8
PR = B // 8
RB = PR // GRID


def _tc_body(gu, gi, gut, git, bu, bi, but, bit, mu, out):
    p = gu[...] * gi[...] + gut[...] * git[...]
    cols = [jnp.sum(p[:, j * K:(j + 1) * K], axis=1, keepdims=True)
            for j in range(8)]
    s = jnp.concatenate(cols, axis=1)
    out[...] = s + bu[...] + bi[...] + but[...] + bit[...] + mu[0, 0]


def kernel(gu, gi, gut, git, bu, bi, but, bit, Mu):
    g1 = [x.reshape(PR, 8 * K) for x in (gu, gi, gut, git)]
    b2 = [x.reshape(PR, 8) for x in (bu, bi, but, bit)]
    gamma_spec = pl.BlockSpec((RB, 8 * K), lambda i: (i, 0))
    bias_spec = pl.BlockSpec((RB, 8), lambda i: (i, 0))
    mu_spec = pl.BlockSpec((1, 1), lambda i: (0, 0))
    out = pl.pallas_call(
        _tc_body,
        grid=(GRID,),
        in_specs=[gamma_spec] * 4 + [bias_spec] * 4 + [mu_spec],
        out_specs=pl.BlockSpec((RB, 8), lambda i: (i, 0)),
        out_shape=jax.ShapeDtypeStruct((PR, 8), jnp.float32),
    )(*g1, *b2, Mu)
    return out.reshape(B)
